# baseline (speedup 1.0000x reference)
import functools

import jax
import jax.numpy as jnp
from jax import lax
from jax.experimental import pallas as pl
from jax.experimental.pallas import tpu as pltpu
from jax.experimental.pallas import tpu_sc as plsc

N = 10000
NP = 10240
E = 320000
D = 128
C = 40
CP = 128

NC = 2
NS = 16
NW = NC * NS
B = 128
CH = 80
EP = NW * CH * B
RPT = NP // NS
_CHH = 8
CH0 = 160
CH1 = 0

_ROWBLK = 512

_mesh = plsc.VectorSubcoreMesh(core_axis_name="c", subcore_axis_name="s")


def _deg_body(src_hbm, dst_hbm, outs_hbm, outd_hbm, idx_v, rows, acc):
    c = lax.axis_index("c")
    s = lax.axis_index("s")
    wid = c * NS + s
    base = s * RPT

    def zero_rows(i, _):
        for k in range(D // 16):
            rows[i, pl.ds(k * 16, 16)] = jnp.zeros((16,), jnp.float32)
        return 0

    def one_rows(i, _):
        for k in range(D // 16):
            rows[i, pl.ds(k * 16, 16)] = jnp.ones((16,), jnp.float32)
        return 0

    def hist_phase(edges_hbm, out_hbm):
        lax.fori_loop(0, B, zero_rows, 0)
        for k in range(RPT // B):
            pltpu.sync_copy(rows, acc.at[pl.ds(base + k * B, B)])
        plsc.subcore_barrier()
        lax.fori_loop(0, B, one_rows, 0)
        pltpu.sync_copy(edges_hbm.at[pl.ds(wid * CH, CH)], idx_v)

        def body(j, _):
            pltpu.sync_copy(rows, acc.at[idx_v.at[j]], add=True)
            return 0

        lax.fori_loop(0, CH, body, 0)
        plsc.subcore_barrier()
        pltpu.sync_copy(acc.at[pl.ds(base, RPT)],
                        out_hbm.at[c, pl.ds(base, RPT)])
        plsc.subcore_barrier()

    hist_phase(src_hbm, outs_hbm)
    hist_phase(dst_hbm, outd_hbm)


_deg_call = pl.kernel(
    _deg_body,
    mesh=_mesh,
    out_type=[
        jax.ShapeDtypeStruct((NC, NP, D), jnp.float32),
        jax.ShapeDtypeStruct((NC, NP, D), jnp.float32),
    ],
    scratch_types=[
        pltpu.VMEM((CH, B), jnp.int32),
        pltpu.VMEM((B, D), jnp.float32),
        pltpu.VMEM_SHARED((NP, D), jnp.float32),
    ],
)


_SOLO = CH1 == 0
_NPART = 1 if _SOLO else NC


def _agg_body(h_hbm, src_hbm, dst_hbm, out_hbm,
              idx_s, idx_d, rows0, rows1, acc, gsem, ssem):
    c = lax.axis_index("c")
    s = lax.axis_index("s")
    w = rows0.shape[-1]

    def work():
        def zfill(i, _):
            for k in range(w // 16):
                rows0[i, pl.ds(k * 16, 16)] = jnp.zeros((16,), jnp.float32)
            return 0

        lax.fori_loop(0, B, zfill, 0)

        base = s * RPT
        for k in range(RPT // B):
            pltpu.sync_copy(rows0, acc.at[pl.ds(base + k * B, B)])
        plsc.subcore_barrier()

        if _SOLO:
            off = s * CH0
            ngrp = CH0 // _CHH
        else:
            off = jnp.where(c == 0, s * CH0, NS * CH0 + s * CH1)
            ngrp = jnp.where(c == 0, CH0 // _CHH, CH1 // _CHH)

        def group(h, _):
            pltpu.sync_copy(src_hbm.at[pl.ds(off + h * _CHH, _CHH)], idx_s)
            pltpu.sync_copy(dst_hbm.at[pl.ds(off + h * _CHH, _CHH)], idx_d)
            for q in range(_CHH // 2):
                j0 = 2 * q
                j1 = 2 * q + 1
                g0 = pltpu.async_copy(h_hbm.at[idx_s.at[j0]], rows0, gsem)
                g1 = pltpu.async_copy(h_hbm.at[idx_s.at[j1]], rows1, gsem)
                g0.wait()
                s0 = pltpu.async_copy(rows0, acc.at[idx_d.at[j0]], ssem,
                                      add=True)
                g1.wait()
                s1 = pltpu.async_copy(rows1, acc.at[idx_d.at[j1]], ssem,
                                      add=True)
                s0.wait()
                s1.wait()
            return 0

        lax.fori_loop(0, ngrp, group, 0)
        plsc.subcore_barrier()

        pltpu.sync_copy(acc.at[pl.ds(base, RPT)],
                        out_hbm.at[c, pl.ds(base, RPT)])

    if _SOLO:
        @pl.when(c == 0)
        def _():
            work()
    else:
        work()


def _make_agg_call(width):
    return pl.kernel(
        _agg_body,
        mesh=_mesh,
        out_type=jax.ShapeDtypeStruct((_NPART, NP, width), jnp.float32),
        scratch_types=[
            pltpu.VMEM((_CHH, B), jnp.int32),
            pltpu.VMEM((_CHH, B), jnp.int32),
            pltpu.VMEM((B, width), jnp.float32),
            pltpu.VMEM((B, width), jnp.float32),
            pltpu.VMEM_SHARED((NP, width), jnp.float32),
            pltpu.SemaphoreType.DMA,
            pltpu.SemaphoreType.DMA,
        ],
    )


_agg_call_d = _make_agg_call(D)
_agg_call_c = _agg_call_d if CP == D else _make_agg_call(CP)


def _norm_col(deg_ref):
    d = jnp.sum(deg_ref[...], axis=0)
    return lax.rsqrt(jnp.maximum(d, 1.0))


def _tc1_body(degs_ref, x_ref, w_ref, out_ref):
    nsrc = _norm_col(degs_ref)
    out_ref[...] = jnp.dot(
        x_ref[...], w_ref[...], preferred_element_type=jnp.float32) * nsrc


_tc1_call = pl.pallas_call(
    _tc1_body,
    grid=(NP // _ROWBLK,),
    in_specs=[
        pl.BlockSpec((NC, _ROWBLK, 1), lambda i: (0, i, 0)),
        pl.BlockSpec((_ROWBLK, D), lambda i: (i, 0)),
        pl.BlockSpec((D, D), lambda i: (0, 0)),
    ],
    out_specs=pl.BlockSpec((_ROWBLK, D), lambda i: (i, 0)),
    out_shape=jax.ShapeDtypeStruct((NP, D), jnp.float32),
)


def _tc2_body(p_ref, degd_ref, degs_ref, b1_ref, w2_ref, out_ref):
    ndst = _norm_col(degd_ref)
    nsrc = _norm_col(degs_ref)
    z = (p_ref[0] + p_ref[1]) * ndst + b1_ref[...]
    z = jnp.maximum(z, 0.0)
    out_ref[...] = jnp.dot(
        z, w2_ref[...], preferred_element_type=jnp.float32) * nsrc


_tc2_call = pl.pallas_call(
    _tc2_body,
    grid=(NP // _ROWBLK,),
    in_specs=[
        pl.BlockSpec((_NPART, _ROWBLK, D), lambda i: (0, i, 0)),
        pl.BlockSpec((NC, _ROWBLK, 1), lambda i: (0, i, 0)),
        pl.BlockSpec((NC, _ROWBLK, 1), lambda i: (0, i, 0)),
        pl.BlockSpec((1, D), lambda i: (0, 0)),
        pl.BlockSpec((D, CP), lambda i: (0, 0)),
    ],
    out_specs=pl.BlockSpec((_ROWBLK, CP), lambda i: (i, 0)),
    out_shape=jax.ShapeDtypeStruct((NP, CP), jnp.float32),
)


def _tc3_body(p_ref, degd_ref, b2_ref, out_ref):
    ndst = _norm_col(degd_ref)
    z = (p_ref[0] + p_ref[1]) * ndst + b2_ref[...]
    col = lax.broadcasted_iota(jnp.int32, z.shape, 1)
    mask = col < C
    m = jnp.max(jnp.where(mask, z, -jnp.inf), axis=1, keepdims=True)
    e = jnp.where(mask, jnp.exp(z - m), 0.0)
    lse = jnp.log(jnp.sum(e, axis=1, keepdims=True))
    out_ref[...] = z - m - lse


_tc3_call = pl.pallas_call(
    _tc3_body,
    grid=(NP // _ROWBLK,),
    in_specs=[
        pl.BlockSpec((_NPART, _ROWBLK, CP), lambda i: (0, i, 0)),
        pl.BlockSpec((NC, _ROWBLK, 1), lambda i: (0, i, 0)),
        pl.BlockSpec((1, CP), lambda i: (0, 0)),
    ],
    out_specs=pl.BlockSpec((_ROWBLK, CP), lambda i: (i, 0)),
    out_shape=jax.ShapeDtypeStruct((NP, CP), jnp.float32),
)


def kernel(x, edge_index, W1, b1, W2, b2):
    src = edge_index[0]
    dst = edge_index[1]
    pad_e = EP - E
    srcp = jnp.concatenate(
        [src, jnp.full((pad_e,), N, jnp.int32)]).reshape(NW * CH, B)
    dstp = jnp.concatenate(
        [dst, jnp.full((pad_e,), N, jnp.int32)]).reshape(NW * CH, B)
    xp = jnp.pad(x, ((0, NP - N), (0, 0)))
    w2p = jnp.pad(W2, ((0, 0), (0, CP - C)))
    b1r = b1.reshape(1, D)
    b2r = jnp.pad(b2, (0, CP - C)).reshape(1, CP)

    degs_f, degd_f = _deg_call(srcp, dstp)
    degs = degs_f[:, :, :1]
    degd = degd_f[:, :, :1]
    h1 = _tc1_call(degs, xp, W1)
    p1 = _agg_call_d(h1, srcp, dstp)
    h2 = _tc2_call(p1, degd, degs, b1r, w2p)
    p2 = _agg_call_c(h2, srcp, dstp)
    outp = _tc3_call(p2, degd, b2r)
    return outp[:N, :C]


# speedup vs baseline: 1.4724x; 1.4724x over previous
import functools

import jax
import jax.numpy as jnp
from jax import lax
from jax.experimental import pallas as pl
from jax.experimental.pallas import tpu as pltpu
from jax.experimental.pallas import tpu_sc as plsc

N = 10000
NP = 10240
E = 320000
D = 128
C = 40
CP = 128

NC = 2
NS = 16
NW = NC * NS
B = 128
CH = 80
EP = NW * CH * B
RPT = NP // NS
_CHH = 8
CH0 = 152
CH1 = 8

_ROWBLK = 512

_mesh = plsc.VectorSubcoreMesh(core_axis_name="c", subcore_axis_name="s")


def _deg_body(src_hbm, dst_hbm, outs_hbm, outd_hbm, idx_v, rows, acc):
    c = lax.axis_index("c")
    s = lax.axis_index("s")
    wid = c * NS + s
    base = s * RPT

    def zero_rows(i, _):
        for k in range(D // 16):
            rows[i, pl.ds(k * 16, 16)] = jnp.zeros((16,), jnp.float32)
        return 0

    def one_rows(i, _):
        for k in range(D // 16):
            rows[i, pl.ds(k * 16, 16)] = jnp.ones((16,), jnp.float32)
        return 0

    def hist_phase(edges_hbm, out_hbm):
        lax.fori_loop(0, B, zero_rows, 0)
        for k in range(RPT // B):
            pltpu.sync_copy(rows, acc.at[pl.ds(base + k * B, B)])
        plsc.subcore_barrier()
        lax.fori_loop(0, B, one_rows, 0)
        pltpu.sync_copy(edges_hbm.at[pl.ds(wid * CH, CH)], idx_v)

        def body(j, _):
            pltpu.sync_copy(rows, acc.at[idx_v.at[j]], add=True)
            return 0

        lax.fori_loop(0, CH, body, 0)
        plsc.subcore_barrier()
        pltpu.sync_copy(acc.at[pl.ds(base, RPT)],
                        out_hbm.at[c, pl.ds(base, RPT)])
        plsc.subcore_barrier()

    hist_phase(src_hbm, outs_hbm)
    hist_phase(dst_hbm, outd_hbm)


_deg_call = pl.kernel(
    _deg_body,
    mesh=_mesh,
    out_type=[
        jax.ShapeDtypeStruct((NC, NP, D), jnp.float32),
        jax.ShapeDtypeStruct((NC, NP, D), jnp.float32),
    ],
    scratch_types=[
        pltpu.VMEM((CH, B), jnp.int32),
        pltpu.VMEM((B, D), jnp.float32),
        pltpu.VMEM_SHARED((NP, D), jnp.float32),
    ],
)


def _agg_body(h_hbm, src_hbm, dst_hbm, out_hbm,
              idx_s, idx_d, rows0, rows1, acc, gsem, ssem):
    c = lax.axis_index("c")
    s = lax.axis_index("s")
    w = rows0.shape[-1]

    def zfill(i, _):
            for k in range(w // 16):
                rows0[i, pl.ds(k * 16, 16)] = jnp.zeros((16,), jnp.float32)
            return 0

    lax.fori_loop(0, B, zfill, 0)

    base = s * RPT
    for k in range(RPT // B):
        pltpu.sync_copy(rows0, acc.at[pl.ds(base + k * B, B)])
    plsc.subcore_barrier()

    off = jnp.where(c == 0, s * CH0, NS * CH0 + s * CH1)
    ngrp = jnp.where(c == 0, CH0 // _CHH, CH1 // _CHH)

    def group(h, _):
        pltpu.sync_copy(src_hbm.at[pl.ds(off + h * _CHH, _CHH)], idx_s)
        pltpu.sync_copy(dst_hbm.at[pl.ds(off + h * _CHH, _CHH)], idx_d)
        for q in range(_CHH // 2):
            j0 = 2 * q
            j1 = 2 * q + 1
            g0 = pltpu.async_copy(h_hbm.at[idx_s.at[j0]], rows0, gsem)
            g1 = pltpu.async_copy(h_hbm.at[idx_s.at[j1]], rows1, gsem)
            g0.wait()
            s0 = pltpu.async_copy(rows0, acc.at[idx_d.at[j0]], ssem, add=True)
            g1.wait()
            s1 = pltpu.async_copy(rows1, acc.at[idx_d.at[j1]], ssem, add=True)
            s0.wait()
            s1.wait()
        return 0

    lax.fori_loop(0, ngrp, group, 0)
    plsc.subcore_barrier()

    pltpu.sync_copy(acc.at[pl.ds(base, RPT)], out_hbm.at[c, pl.ds(base, RPT)])


def _make_agg_call(width):
    return pl.kernel(
        _agg_body,
        mesh=_mesh,
        out_type=jax.ShapeDtypeStruct((NC, NP, width), jnp.float32),
        scratch_types=[
            pltpu.VMEM((_CHH, B), jnp.int32),
            pltpu.VMEM((_CHH, B), jnp.int32),
            pltpu.VMEM((B, width), jnp.float32),
            pltpu.VMEM((B, width), jnp.float32),
            pltpu.VMEM_SHARED((NP, width), jnp.float32),
            pltpu.SemaphoreType.DMA,
            pltpu.SemaphoreType.DMA,
        ],
    )


_agg_call_d = _make_agg_call(D)
_agg_call_c = _agg_call_d if CP == D else _make_agg_call(CP)


def _norm_col(deg_ref):
    d = jnp.sum(deg_ref[...], axis=0)
    return lax.rsqrt(jnp.maximum(d, 1.0))


def _tc1_body(degs_ref, x_ref, w_ref, out_ref):
    nsrc = _norm_col(degs_ref)
    out_ref[...] = jnp.dot(
        x_ref[...], w_ref[...], preferred_element_type=jnp.float32) * nsrc


_tc1_call = pl.pallas_call(
    _tc1_body,
    grid=(NP // _ROWBLK,),
    in_specs=[
        pl.BlockSpec((NC, _ROWBLK, 1), lambda i: (0, i, 0)),
        pl.BlockSpec((_ROWBLK, D), lambda i: (i, 0)),
        pl.BlockSpec((D, D), lambda i: (0, 0)),
    ],
    out_specs=pl.BlockSpec((_ROWBLK, D), lambda i: (i, 0)),
    out_shape=jax.ShapeDtypeStruct((NP, D), jnp.float32),
)


def _tc2_body(p_ref, degd_ref, degs_ref, b1_ref, w2_ref, out_ref):
    ndst = _norm_col(degd_ref)
    nsrc = _norm_col(degs_ref)
    z = (p_ref[0] + p_ref[1]) * ndst + b1_ref[...]
    z = jnp.maximum(z, 0.0)
    out_ref[...] = jnp.dot(
        z, w2_ref[...], preferred_element_type=jnp.float32) * nsrc


_tc2_call = pl.pallas_call(
    _tc2_body,
    grid=(NP // _ROWBLK,),
    in_specs=[
        pl.BlockSpec((NC, _ROWBLK, D), lambda i: (0, i, 0)),
        pl.BlockSpec((NC, _ROWBLK, 1), lambda i: (0, i, 0)),
        pl.BlockSpec((NC, _ROWBLK, 1), lambda i: (0, i, 0)),
        pl.BlockSpec((1, D), lambda i: (0, 0)),
        pl.BlockSpec((D, CP), lambda i: (0, 0)),
    ],
    out_specs=pl.BlockSpec((_ROWBLK, CP), lambda i: (i, 0)),
    out_shape=jax.ShapeDtypeStruct((NP, CP), jnp.float32),
)


def _tc3_body(p_ref, degd_ref, b2_ref, out_ref):
    ndst = _norm_col(degd_ref)
    z = (p_ref[0] + p_ref[1]) * ndst + b2_ref[...]
    col = lax.broadcasted_iota(jnp.int32, z.shape, 1)
    mask = col < C
    m = jnp.max(jnp.where(mask, z, -jnp.inf), axis=1, keepdims=True)
    e = jnp.where(mask, jnp.exp(z - m), 0.0)
    lse = jnp.log(jnp.sum(e, axis=1, keepdims=True))
    out_ref[...] = z - m - lse


_tc3_call = pl.pallas_call(
    _tc3_body,
    grid=(NP // _ROWBLK,),
    in_specs=[
        pl.BlockSpec((NC, _ROWBLK, CP), lambda i: (0, i, 0)),
        pl.BlockSpec((NC, _ROWBLK, 1), lambda i: (0, i, 0)),
        pl.BlockSpec((1, CP), lambda i: (0, 0)),
    ],
    out_specs=pl.BlockSpec((_ROWBLK, CP), lambda i: (i, 0)),
    out_shape=jax.ShapeDtypeStruct((NP, CP), jnp.float32),
)


def kernel(x, edge_index, W1, b1, W2, b2):
    src = edge_index[0]
    dst = edge_index[1]
    pad_e = EP - E
    srcp = jnp.concatenate(
        [src, jnp.full((pad_e,), N, jnp.int32)]).reshape(NW * CH, B)
    dstp = jnp.concatenate(
        [dst, jnp.full((pad_e,), N, jnp.int32)]).reshape(NW * CH, B)
    xp = jnp.pad(x, ((0, NP - N), (0, 0)))
    w2p = jnp.pad(W2, ((0, 0), (0, CP - C)))
    b1r = b1.reshape(1, D)
    b2r = jnp.pad(b2, (0, CP - C)).reshape(1, CP)

    degs_f, degd_f = _deg_call(srcp, dstp)
    degs = degs_f[:, :, :1]
    degd = degd_f[:, :, :1]
    h1 = _tc1_call(degs, xp, W1)
    p1 = _agg_call_d(h1, srcp, dstp)
    h2 = _tc2_call(p1, degd, degs, b1r, w2p)
    p2 = _agg_call_c(h2, srcp, dstp)
    outp = _tc3_call(p2, degd, b2r)
    return outp[:N, :C]
